# Initial kernel scaffold; baseline (speedup 1.0000x reference)
import jax
import jax.numpy as jnp
from jax.experimental import pallas as pl


def kernel(x, pe):
    raise NotImplementedError("write your pallas kernel here")


# speedup vs baseline: 2.9822x; 2.9822x over previous
import functools

import jax
import jax.numpy as jnp
from jax.experimental import pallas as pl


def _body(x_ref, pe_a, pe_b, o_ref):
    rows = jnp.concatenate([pe_a[1:, :], pe_b[:1, :]], axis=0)
    o_ref[...] = x_ref[...] + rows[None, :, :]


def kernel(x, pe):
    b, l, d = x.shape
    cl = 256
    return pl.pallas_call(
        _body,
        grid=(l // cl,),
        in_specs=[
            pl.BlockSpec((b, cl, d), lambda i: (0, i, 0)),
            pl.BlockSpec((cl, d), lambda i: (i, 0)),
            pl.BlockSpec((cl, d), lambda i: (i + 1, 0)),
        ],
        out_specs=pl.BlockSpec((b, cl, d), lambda i: (0, i, 0)),
        out_shape=jax.ShapeDtypeStruct((b, l, d), x.dtype),
    )(x, pe, pe)
